# baseline (speedup 1.0000x reference)
import functools
import math

import jax
import jax.numpy as jnp
from jax import lax
from jax.experimental import pallas as pl
from jax.experimental.pallas import tpu as pltpu
from jax.experimental.pallas import tpu_sc as plsc

VOCAB = 30522
H = 768
S = 512
B = 64
NC = 2
NS = 16
NW = NC * NS
SPW = S // NW
BS = 16
EPS = 1e-12
SQRT_H = math.sqrt(float(H))


@functools.partial(
    pl.kernel,
    out_type=jax.ShapeDtypeStruct((S, B, H), jnp.float32),
    mesh=plsc.VectorSubcoreMesh(core_axis_name="c", subcore_axis_name="s"),
    scratch_types=[
        pltpu.VMEM((SPW, B), jnp.int32),
        pltpu.VMEM((B, H), jnp.float32),
        pltpu.VMEM((B, H), jnp.float32),
        pltpu.SemaphoreType.DMA,
        pltpu.SemaphoreType.DMA,
        pltpu.SemaphoreType.DMA,
        pltpu.SemaphoreType.DMA,
    ],
    compiler_params=pltpu.CompilerParams(needs_layout_passes=False),
)
def _gather_kernel(xt, word, out, idx_v, buf0, buf1, sg0, sg1, ss0, ss1):
    w = lax.axis_index("s") * NC + lax.axis_index("c")
    s0 = w * SPW

    pltpu.sync_copy(xt.at[pl.ds(s0, SPW)], idx_v)

    bufs = (buf0, buf1)
    gsems = (sg0, sg1)
    ssems = (ss0, ss1)

    pltpu.async_copy(word.at[idx_v.at[0]], buf0, sg0)

    def _giter(g, _):
        for par in range(2):
            c = g * 2 + par
            buf = bufs[par]
            obuf = bufs[1 - par]

            @pl.when(c > 0)
            def _():
                pltpu.make_async_copy(obuf, out.at[s0], ssems[1 - par]).wait()

            @pl.when(c + 1 < SPW)
            def _():
                pltpu.async_copy(word.at[idx_v.at[c + 1]], obuf, gsems[1 - par])

            pltpu.make_async_copy(word.at[pl.ds(0, B)], buf, gsems[par]).wait()
            pltpu.async_copy(buf, out.at[s0 + c], ssems[par])
        return 0

    lax.fori_loop(0, SPW // 2, _giter, 0)
    pltpu.make_async_copy(buf1, out.at[s0], ss1).wait()


def _ln_body(scr, pos, typ, gamma, beta, out):
    e = scr[...] + pos[...][:, None, :] + typ[...][0][None, None, :]
    mean = jnp.mean(e, axis=-1, keepdims=True)
    var = jnp.mean(jnp.square(e - mean), axis=-1, keepdims=True)
    h = (e - mean) * lax.rsqrt(var + EPS)
    out[...] = (h * gamma[...][0] + beta[...][0]) * SQRT_H


_ln_kernel = pl.pallas_call(
    _ln_body,
    grid=(S // BS,),
    in_specs=[
        pl.BlockSpec((BS, B, H), lambda i: (i, 0, 0)),
        pl.BlockSpec((BS, H), lambda i: (i, 0)),
        pl.BlockSpec((2, H), lambda i: (0, 0)),
        pl.BlockSpec((1, H), lambda i: (0, 0)),
        pl.BlockSpec((1, H), lambda i: (0, 0)),
    ],
    out_specs=pl.BlockSpec((BS, B, H), lambda i: (i, 0, 0)),
    out_shape=jax.ShapeDtypeStruct((S, B, H), jnp.float32),
    compiler_params=pltpu.CompilerParams(
        dimension_semantics=("arbitrary",),
    ),
)


def kernel(x, word_emb, pos_emb, type_emb, ln_gamma, ln_beta):
    gathered = _gather_kernel(x.T, word_emb)
    return _ln_kernel(gathered, pos_emb, type_emb,
                      ln_gamma.reshape(1, H), ln_beta.reshape(1, H))
